# initial kernel scaffold (unmeasured)
import jax
import jax.numpy as jnp
from jax import lax
from jax.experimental import pallas as pl
from jax.experimental.pallas import tpu as pltpu

N_DEV = 4
B = 4
SQ = 1024
SKV = 1024
DM = 1024
HL = 8
DH = 128
SCALE = 0.08838834764831843
BLK = 64
NEG = -1e9


def _allgather_x(x2d):

    def body(x_ref, out_ref, send_sems, recv_sems):
        my = lax.axis_index("i")

        barrier = pltpu.get_barrier_semaphore()
        for p in range(1, N_DEV):
            peer = lax.rem(my + p, N_DEV)
            pl.semaphore_signal(barrier, inc=1, device_id=(peer,),
                                device_id_type=pl.DeviceIdType.MESH)
        pl.semaphore_wait(barrier, N_DEV - 1)

        out_ref[pl.ds(my, 1)] = x_ref[...][None]

        sends = []
        for p in range(1, N_DEV):
            peer = lax.rem(my + p, N_DEV)
            rdma = pltpu.make_async_remote_copy(
                src_ref=x_ref,
                dst_ref=out_ref.at[my],
                send_sem=send_sems.at[peer],
                recv_sem=recv_sems.at[my],
                device_id=(peer,),
                device_id_type=pl.DeviceIdType.MESH,
            )
            rdma.start()
            sends.append(rdma)

        for p in range(1, N_DEV):
            j = lax.rem(my + p, N_DEV)
            recv = pltpu.make_async_remote_copy(
                src_ref=x_ref,
                dst_ref=out_ref.at[j],
                send_sem=send_sems.at[my],
                recv_sem=recv_sems.at[j],
                device_id=(j,),
                device_id_type=pl.DeviceIdType.MESH,
            )
            recv.wait_recv()
        for rdma in sends:
            rdma.wait_send()

    return pl.pallas_call(
        body,
        out_shape=jax.ShapeDtypeStruct((B, SQ, DM), x2d.dtype),
        in_specs=[pl.BlockSpec(memory_space=pltpu.VMEM)],
        out_specs=pl.BlockSpec(memory_space=pltpu.VMEM),
        scratch_shapes=[
            pltpu.SemaphoreType.DMA((N_DEV,)),
            pltpu.SemaphoreType.DMA((N_DEV,)),
        ],
        compiler_params=pltpu.CompilerParams(collective_id=0),
    )(x2d)


def _compute_partials(x_all, Wq, K_loc, V_loc, Wo):

    def body(x_ref, wq_ref, k_ref, v_ref, wo_ref, out_ref, bias_ref):
        h = pl.program_id(1)
        first = (pl.program_id(0) == 0) & (h == 0)

        @pl.when(first)
        def _():
            r = lax.broadcasted_iota(jnp.int32, (SQ, SKV), 0) // BLK
            c = lax.broadcasted_iota(jnp.int32, (SQ, SKV), 1) // BLK
            mask = (r == c) | (c == 0) | (lax.rem(r + c, 3) == 0)
            bias_ref[...] = jnp.where(mask, 0.0, NEG).astype(jnp.float32)

        q = jnp.dot(x_ref[0], wq_ref[...], preferred_element_type=jnp.float32)
        k = k_ref[0, :, 0, :]
        s = lax.dot_general(q, k, (((1,), (1,)), ((), ())),
                            preferred_element_type=jnp.float32)
        s = s * SCALE + bias_ref[...]
        m = jnp.max(s, axis=1, keepdims=True)
        w = jnp.exp(s - m)
        w = w / jnp.sum(w, axis=1, keepdims=True)
        ctx = jnp.dot(w, v_ref[0, :, 0, :], preferred_element_type=jnp.float32)
        contrib = jnp.dot(ctx, wo_ref[...], preferred_element_type=jnp.float32)

        @pl.when(h == 0)
        def _():
            out_ref[0] = contrib

        @pl.when(h > 0)
        def _():
            out_ref[0] += contrib

    return pl.pallas_call(
        body,
        grid=(B, HL),
        in_specs=[
            pl.BlockSpec((1, SQ, DM), lambda b, h: (b, 0, 0)),
            pl.BlockSpec((DM, DH), lambda b, h: (0, h)),
            pl.BlockSpec((1, SKV, 1, DH), lambda b, h: (b, 0, h, 0)),
            pl.BlockSpec((1, SKV, 1, DH), lambda b, h: (b, 0, h, 0)),
            pl.BlockSpec((DH, DM), lambda b, h: (h, 0)),
        ],
        out_specs=pl.BlockSpec((1, SQ, DM), lambda b, h: (b, 0, 0)),
        out_shape=jax.ShapeDtypeStruct((B, SQ, DM), jnp.float32),
        scratch_shapes=[pltpu.VMEM((SQ, SKV), jnp.float32)],
        compiler_params=pltpu.CompilerParams(
            dimension_semantics=("arbitrary", "arbitrary"),
        ),
    )(x_all, Wq, K_loc, V_loc, Wo)


def _reduce_partials(partials):

    def body(p_ref, out_ref, recv_buf, send_sems, recv_sems):
        my = lax.axis_index("i")

        barrier = pltpu.get_barrier_semaphore()
        for p in range(1, N_DEV):
            peer = lax.rem(my + p, N_DEV)
            pl.semaphore_signal(barrier, inc=1, device_id=(peer,),
                                device_id_type=pl.DeviceIdType.MESH)
        pl.semaphore_wait(barrier, N_DEV - 1)

        sends = []
        for p in range(1, N_DEV):
            peer = lax.rem(my + p, N_DEV)
            rdma = pltpu.make_async_remote_copy(
                src_ref=p_ref.at[peer],
                dst_ref=recv_buf.at[my],
                send_sem=send_sems.at[peer],
                recv_sem=recv_sems.at[my],
                device_id=(peer,),
                device_id_type=pl.DeviceIdType.MESH,
            )
            rdma.start()
            sends.append(rdma)

        acc = p_ref[pl.ds(my, 1)][0]
        for p in range(1, N_DEV):
            j = lax.rem(my + p, N_DEV)
            recv = pltpu.make_async_remote_copy(
                src_ref=p_ref.at[j],
                dst_ref=recv_buf.at[j],
                send_sem=send_sems.at[my],
                recv_sem=recv_sems.at[j],
                device_id=(j,),
                device_id_type=pl.DeviceIdType.MESH,
            )
            recv.wait_recv()
            acc = acc + recv_buf[pl.ds(j, 1)][0]
        out_ref[...] = acc

        for rdma in sends:
            rdma.wait_send()

    return pl.pallas_call(
        body,
        out_shape=jax.ShapeDtypeStruct((SQ, DM), jnp.float32),
        in_specs=[pl.BlockSpec(memory_space=pltpu.VMEM)],
        out_specs=pl.BlockSpec(memory_space=pltpu.VMEM),
        scratch_shapes=[
            pltpu.VMEM((N_DEV, SQ, DM), jnp.float32),
            pltpu.SemaphoreType.DMA((N_DEV,)),
            pltpu.SemaphoreType.DMA((N_DEV,)),
        ],
        compiler_params=pltpu.CompilerParams(collective_id=1),
    )(partials)


def kernel(x, Wq, K_ext, V_ext, Wo):
    my = lax.axis_index("i")
    K_loc = lax.dynamic_slice_in_dim(K_ext, my * HL, HL, axis=2)
    V_loc = lax.dynamic_slice_in_dim(V_ext, my * HL, HL, axis=2)

    x_all = _allgather_x(x[0])
    partials = _compute_partials(x_all, Wq, K_loc, V_loc, Wo)
    out = _reduce_partials(partials)
    return out[None]


# baseline (device time: 440513 ns/iter reference)
import jax
import jax.numpy as jnp
from jax import lax
from jax.experimental import pallas as pl
from jax.experimental.pallas import tpu as pltpu

N_DEV = 4
B = 4
SQ = 1024
SKV = 1024
DM = 1024
HL = 8
DH = 128
SCALE = 0.08838834764831843
BLK = 64
NEG = -1e9


def _allgather_x(x2d):

    def body(x_ref, out_ref, send_sems, recv_sems):
        my = lax.axis_index("i")

        barrier = pltpu.get_barrier_semaphore()
        for p in range(1, N_DEV):
            peer = lax.rem(my + p, N_DEV)
            pl.semaphore_signal(barrier, inc=1, device_id=(peer,),
                                device_id_type=pl.DeviceIdType.MESH)
        pl.semaphore_wait(barrier, N_DEV - 1)

        out_ref[pl.ds(my, 1)] = x_ref[...][None]

        sends = []
        for p in range(1, N_DEV):
            peer = lax.rem(my + p, N_DEV)
            rdma = pltpu.make_async_remote_copy(
                src_ref=x_ref,
                dst_ref=out_ref.at[my],
                send_sem=send_sems.at[peer],
                recv_sem=recv_sems.at[my],
                device_id=(peer,),
                device_id_type=pl.DeviceIdType.MESH,
            )
            rdma.start()
            sends.append(rdma)

        for p in range(1, N_DEV):
            j = lax.rem(my + p, N_DEV)
            recv = pltpu.make_async_remote_copy(
                src_ref=x_ref,
                dst_ref=out_ref.at[j],
                send_sem=send_sems.at[my],
                recv_sem=recv_sems.at[j],
                device_id=(j,),
                device_id_type=pl.DeviceIdType.MESH,
            )
            recv.wait_recv()
        for rdma in sends:
            rdma.wait_send()

    return pl.pallas_call(
        body,
        out_shape=jax.ShapeDtypeStruct((B, SQ, DM), x2d.dtype),
        in_specs=[pl.BlockSpec(memory_space=pltpu.VMEM)],
        out_specs=pl.BlockSpec(memory_space=pltpu.VMEM),
        scratch_shapes=[
            pltpu.SemaphoreType.DMA((N_DEV,)),
            pltpu.SemaphoreType.DMA((N_DEV,)),
        ],
        compiler_params=pltpu.CompilerParams(collective_id=0),
    )(x2d)


def _compute_partials(x_all, Wq, K_loc, V_loc, Wo):

    def body(x_ref, wq_ref, k_ref, v_ref, wo_ref, out_ref, bias_ref):
        h = pl.program_id(1)
        first = (pl.program_id(0) == 0) & (h == 0)

        @pl.when(first)
        def _():
            r = lax.broadcasted_iota(jnp.int32, (SQ, SKV), 0) // BLK
            c = lax.broadcasted_iota(jnp.int32, (SQ, SKV), 1) // BLK
            mask = (r == c) | (c == 0) | (lax.rem(r + c, 3) == 0)
            bias_ref[...] = jnp.where(mask, 0.0, NEG).astype(jnp.float32)

        q = jnp.dot(x_ref[0], wq_ref[...], preferred_element_type=jnp.float32)
        k = k_ref[0, 0]
        s = lax.dot_general(q, k, (((1,), (1,)), ((), ())),
                            preferred_element_type=jnp.float32)
        s = s * SCALE + bias_ref[...]
        m = jnp.max(s, axis=1, keepdims=True)
        w = jnp.exp(s - m)
        w = w / jnp.sum(w, axis=1, keepdims=True)
        ctx = jnp.dot(w, v_ref[0, 0], preferred_element_type=jnp.float32)
        contrib = jnp.dot(ctx, wo_ref[...], preferred_element_type=jnp.float32)

        @pl.when(h == 0)
        def _():
            out_ref[0] = contrib

        @pl.when(h > 0)
        def _():
            out_ref[0] += contrib

    return pl.pallas_call(
        body,
        grid=(B, HL),
        in_specs=[
            pl.BlockSpec((1, SQ, DM), lambda b, h: (b, 0, 0)),
            pl.BlockSpec((DM, DH), lambda b, h: (0, h)),
            pl.BlockSpec((1, 1, SKV, DH), lambda b, h: (b, h, 0, 0)),
            pl.BlockSpec((1, 1, SKV, DH), lambda b, h: (b, h, 0, 0)),
            pl.BlockSpec((DH, DM), lambda b, h: (h, 0)),
        ],
        out_specs=pl.BlockSpec((1, SQ, DM), lambda b, h: (b, 0, 0)),
        out_shape=jax.ShapeDtypeStruct((B, SQ, DM), jnp.float32),
        scratch_shapes=[pltpu.VMEM((SQ, SKV), jnp.float32)],
        compiler_params=pltpu.CompilerParams(
            dimension_semantics=("arbitrary", "arbitrary"),
        ),
    )(x_all, Wq, K_loc, V_loc, Wo)


def _reduce_partials(partials):

    def body(p_ref, out_ref, recv_buf, send_sems, recv_sems):
        my = lax.axis_index("i")

        barrier = pltpu.get_barrier_semaphore()
        for p in range(1, N_DEV):
            peer = lax.rem(my + p, N_DEV)
            pl.semaphore_signal(barrier, inc=1, device_id=(peer,),
                                device_id_type=pl.DeviceIdType.MESH)
        pl.semaphore_wait(barrier, N_DEV - 1)

        sends = []
        for p in range(1, N_DEV):
            peer = lax.rem(my + p, N_DEV)
            rdma = pltpu.make_async_remote_copy(
                src_ref=p_ref.at[peer],
                dst_ref=recv_buf.at[my],
                send_sem=send_sems.at[peer],
                recv_sem=recv_sems.at[my],
                device_id=(peer,),
                device_id_type=pl.DeviceIdType.MESH,
            )
            rdma.start()
            sends.append(rdma)

        acc = p_ref[pl.ds(my, 1)][0]
        for p in range(1, N_DEV):
            j = lax.rem(my + p, N_DEV)
            recv = pltpu.make_async_remote_copy(
                src_ref=p_ref.at[j],
                dst_ref=recv_buf.at[j],
                send_sem=send_sems.at[my],
                recv_sem=recv_sems.at[j],
                device_id=(j,),
                device_id_type=pl.DeviceIdType.MESH,
            )
            recv.wait_recv()
            acc = acc + recv_buf[pl.ds(j, 1)][0]
        out_ref[...] = acc

        for rdma in sends:
            rdma.wait_send()

    return pl.pallas_call(
        body,
        out_shape=jax.ShapeDtypeStruct((SQ, DM), jnp.float32),
        in_specs=[pl.BlockSpec(memory_space=pltpu.VMEM)],
        out_specs=pl.BlockSpec(memory_space=pltpu.VMEM),
        scratch_shapes=[
            pltpu.VMEM((N_DEV, SQ, DM), jnp.float32),
            pltpu.SemaphoreType.DMA((N_DEV,)),
            pltpu.SemaphoreType.DMA((N_DEV,)),
        ],
        compiler_params=pltpu.CompilerParams(collective_id=1),
    )(partials)


def kernel(x, Wq, K_ext, V_ext, Wo):
    my = lax.axis_index("i")
    K_loc = lax.dynamic_slice_in_dim(K_ext, my * HL, HL, axis=2)
    V_loc = lax.dynamic_slice_in_dim(V_ext, my * HL, HL, axis=2)
    K_loc = jnp.transpose(K_loc, (0, 2, 1, 3))
    V_loc = jnp.transpose(V_loc, (0, 2, 1, 3))

    x_all = _allgather_x(x[0])
    partials = _compute_partials(x_all, Wq, K_loc, V_loc, Wo)
    out = _reduce_partials(partials)
    return out[None]


# device time: 310421 ns/iter; 1.4191x vs baseline; 1.4191x over previous
import jax
import jax.numpy as jnp
from jax import lax
from jax.experimental import pallas as pl
from jax.experimental.pallas import tpu as pltpu

N_DEV = 4
B = 4
SQ = 1024
SKV = 1024
DM = 1024
HL = 8
DH = 128
SCALE = 0.08838834764831843
BLK = 64
NEG = -1e9
BF = jnp.bfloat16


def _allgather_x(x2d):

    def body(x_ref, out_ref, send_sems, recv_sems):
        my = lax.axis_index("i")

        barrier = pltpu.get_barrier_semaphore()
        for p in range(1, N_DEV):
            peer = lax.rem(my + p, N_DEV)
            pl.semaphore_signal(barrier, inc=1, device_id=(peer,),
                                device_id_type=pl.DeviceIdType.MESH)
        pl.semaphore_wait(barrier, N_DEV - 1)

        out_ref[pl.ds(my, 1)] = x_ref[...].astype(BF)[None]

        sends = []
        for p in range(1, N_DEV):
            peer = lax.rem(my + p, N_DEV)
            rdma = pltpu.make_async_remote_copy(
                src_ref=out_ref.at[my],
                dst_ref=out_ref.at[my],
                send_sem=send_sems.at[peer],
                recv_sem=recv_sems.at[my],
                device_id=(peer,),
                device_id_type=pl.DeviceIdType.MESH,
            )
            rdma.start()
            sends.append(rdma)

        for p in range(1, N_DEV):
            j = lax.rem(my + p, N_DEV)
            recv = pltpu.make_async_remote_copy(
                src_ref=out_ref.at[my],
                dst_ref=out_ref.at[j],
                send_sem=send_sems.at[my],
                recv_sem=recv_sems.at[j],
                device_id=(j,),
                device_id_type=pl.DeviceIdType.MESH,
            )
            recv.wait_recv()
        for rdma in sends:
            rdma.wait_send()

    return pl.pallas_call(
        body,
        out_shape=jax.ShapeDtypeStruct((B, SQ, DM), BF),
        in_specs=[pl.BlockSpec(memory_space=pltpu.VMEM)],
        out_specs=pl.BlockSpec(memory_space=pltpu.VMEM),
        scratch_shapes=[
            pltpu.SemaphoreType.DMA((N_DEV,)),
            pltpu.SemaphoreType.DMA((N_DEV,)),
        ],
        compiler_params=pltpu.CompilerParams(collective_id=0),
    )(x2d)


def _compute_partials(x_all, Wq, K_ext, V_ext, Wo):

    def body(x_ref, wq_ref, k_ref, v_ref, wo_ref, out_ref,
             k_st, v_st, acc, bias, dma_sem):
        b = pl.program_id(0)
        h = pl.program_id(1)
        my = lax.axis_index("i")

        @pl.when((b == 0) & (h == 0))
        def _():
            r = lax.broadcasted_iota(jnp.int32, (SQ, SKV), 0) // BLK
            c = lax.broadcasted_iota(jnp.int32, (SQ, SKV), 1) // BLK
            mask = (r == c) | (c == 0) | (lax.rem(r + c, 3) == 0)
            bias[...] = jnp.where(mask, 0.0, NEG).astype(jnp.float32)

        @pl.when(h == 0)
        def _():
            cpk = pltpu.make_async_copy(
                k_ref.at[b, :, pl.ds(my * HL, HL), :], k_st, dma_sem)
            cpk.start()
            cpk.wait()
            cpv = pltpu.make_async_copy(
                v_ref.at[b, :, pl.ds(my * HL, HL), :], v_st, dma_sem)
            cpv.start()
            cpv.wait()

        q16 = jnp.dot(x_ref[0], wq_ref[...].astype(BF),
                      preferred_element_type=jnp.float32).astype(BF)
        k16 = k_st[:, h, :].astype(BF)
        s = lax.dot_general(q16, k16, (((1,), (1,)), ((), ())),
                            preferred_element_type=jnp.float32)
        s = s * SCALE + bias[...]
        m = jnp.max(s, axis=1, keepdims=True)
        w = jnp.exp(s - m)
        recip = 1.0 / jnp.sum(w, axis=1, keepdims=True)
        w16 = (w * recip).astype(BF)
        v16 = v_st[:, h, :].astype(BF)
        ctx16 = jnp.dot(w16, v16, preferred_element_type=jnp.float32).astype(BF)
        contrib = jnp.dot(ctx16, wo_ref[...].astype(BF),
                          preferred_element_type=jnp.float32)

        @pl.when(h == 0)
        def _():
            acc[...] = contrib

        @pl.when(h > 0)
        def _():
            acc[...] += contrib

        @pl.when(h == HL - 1)
        def _():
            out_ref[0] = acc[...].astype(BF)

    return pl.pallas_call(
        body,
        grid=(B, HL),
        in_specs=[
            pl.BlockSpec((1, SQ, DM), lambda b, h: (b, 0, 0)),
            pl.BlockSpec((DM, DH), lambda b, h: (0, h)),
            pl.BlockSpec(memory_space=pltpu.HBM),
            pl.BlockSpec(memory_space=pltpu.HBM),
            pl.BlockSpec((DH, DM), lambda b, h: (h, 0)),
        ],
        out_specs=pl.BlockSpec((1, SQ, DM), lambda b, h: (b, 0, 0)),
        out_shape=jax.ShapeDtypeStruct((B, SQ, DM), BF),
        scratch_shapes=[
            pltpu.VMEM((SKV, HL, DH), jnp.float32),
            pltpu.VMEM((SKV, HL, DH), jnp.float32),
            pltpu.VMEM((SQ, DM), jnp.float32),
            pltpu.VMEM((SQ, SKV), jnp.float32),
            pltpu.SemaphoreType.DMA,
        ],
        compiler_params=pltpu.CompilerParams(
            dimension_semantics=("arbitrary", "arbitrary"),
        ),
    )(x_all, Wq, K_ext, V_ext, Wo)


def _reduce_partials(partials):

    def body(p_ref, out_ref, recv_buf, send_sems, recv_sems):
        my = lax.axis_index("i")

        barrier = pltpu.get_barrier_semaphore()
        for p in range(1, N_DEV):
            peer = lax.rem(my + p, N_DEV)
            pl.semaphore_signal(barrier, inc=1, device_id=(peer,),
                                device_id_type=pl.DeviceIdType.MESH)
        pl.semaphore_wait(barrier, N_DEV - 1)

        sends = []
        for p in range(1, N_DEV):
            peer = lax.rem(my + p, N_DEV)
            rdma = pltpu.make_async_remote_copy(
                src_ref=p_ref.at[peer],
                dst_ref=recv_buf.at[my],
                send_sem=send_sems.at[peer],
                recv_sem=recv_sems.at[my],
                device_id=(peer,),
                device_id_type=pl.DeviceIdType.MESH,
            )
            rdma.start()
            sends.append(rdma)

        acc = p_ref[pl.ds(my, 1)][0].astype(jnp.float32)
        for p in range(1, N_DEV):
            j = lax.rem(my + p, N_DEV)
            recv = pltpu.make_async_remote_copy(
                src_ref=p_ref.at[j],
                dst_ref=recv_buf.at[j],
                send_sem=send_sems.at[my],
                recv_sem=recv_sems.at[j],
                device_id=(j,),
                device_id_type=pl.DeviceIdType.MESH,
            )
            recv.wait_recv()
            acc = acc + recv_buf[pl.ds(j, 1)][0].astype(jnp.float32)
        out_ref[...] = acc

        for rdma in sends:
            rdma.wait_send()

    return pl.pallas_call(
        body,
        out_shape=jax.ShapeDtypeStruct((SQ, DM), jnp.float32),
        in_specs=[pl.BlockSpec(memory_space=pltpu.VMEM)],
        out_specs=pl.BlockSpec(memory_space=pltpu.VMEM),
        scratch_shapes=[
            pltpu.VMEM((N_DEV, SQ, DM), BF),
            pltpu.SemaphoreType.DMA((N_DEV,)),
            pltpu.SemaphoreType.DMA((N_DEV,)),
        ],
        compiler_params=pltpu.CompilerParams(collective_id=1),
    )(partials)


def kernel(x, Wq, K_ext, V_ext, Wo):
    x_all = _allgather_x(x[0])
    partials = _compute_partials(x_all, Wq, K_ext, V_ext, Wo)
    out = _reduce_partials(partials)
    return out[None]


# device time: 240695 ns/iter; 1.8302x vs baseline; 1.2897x over previous
import jax
import jax.numpy as jnp
from jax import lax
from jax.experimental import pallas as pl
from jax.experimental.pallas import tpu as pltpu

N_DEV = 4
B = 4
SQ = 1024
SKV = 1024
DM = 1024
HL = 8
DH = 128
SCALE = 0.08838834764831843
BLK = 64
NEG = -1e9
BF = jnp.bfloat16


def _allgather_x(x2d):

    def body(x_ref, out_ref, send_sems, recv_sems):
        my = lax.axis_index("i")

        barrier = pltpu.get_barrier_semaphore()
        for p in range(1, N_DEV):
            peer = lax.rem(my + p, N_DEV)
            pl.semaphore_signal(barrier, inc=1, device_id=(peer,),
                                device_id_type=pl.DeviceIdType.MESH)
        pl.semaphore_wait(barrier, N_DEV - 1)

        out_ref[pl.ds(my, 1)] = x_ref[...].astype(BF)[None]

        sends = []
        for p in range(1, N_DEV):
            peer = lax.rem(my + p, N_DEV)
            rdma = pltpu.make_async_remote_copy(
                src_ref=out_ref.at[my],
                dst_ref=out_ref.at[my],
                send_sem=send_sems.at[peer],
                recv_sem=recv_sems.at[my],
                device_id=(peer,),
                device_id_type=pl.DeviceIdType.MESH,
            )
            rdma.start()
            sends.append(rdma)

        for p in range(1, N_DEV):
            j = lax.rem(my + p, N_DEV)
            recv = pltpu.make_async_remote_copy(
                src_ref=out_ref.at[my],
                dst_ref=out_ref.at[j],
                send_sem=send_sems.at[my],
                recv_sem=recv_sems.at[j],
                device_id=(j,),
                device_id_type=pl.DeviceIdType.MESH,
            )
            recv.wait_recv()
        for rdma in sends:
            rdma.wait_send()

    return pl.pallas_call(
        body,
        out_shape=jax.ShapeDtypeStruct((B, SQ, DM), BF),
        in_specs=[pl.BlockSpec(memory_space=pltpu.VMEM)],
        out_specs=pl.BlockSpec(memory_space=pltpu.VMEM),
        scratch_shapes=[
            pltpu.SemaphoreType.DMA((N_DEV,)),
            pltpu.SemaphoreType.DMA((N_DEV,)),
        ],
        compiler_params=pltpu.CompilerParams(collective_id=0),
    )(x2d)


def _compute_partials(x_all, Wq, K_ext, V_ext, Wo):

    def body(x_ref, wq_ref, k_ref, v_ref, wo_ref, out_ref,
             k_st, v_st, q16, ctx16, wq16, wo16, bias, dma_sem):
        b = pl.program_id(0)
        h = pl.program_id(1)
        my = lax.axis_index("i")

        @pl.when((b == 0) & (h == 0))
        def _():
            r = lax.broadcasted_iota(jnp.int32, (SQ, SKV), 0) // BLK
            c = lax.broadcasted_iota(jnp.int32, (SQ, SKV), 1) // BLK
            mask = (r == c) | (c == 0) | (lax.rem(r + c, 3) == 0)
            bias[...] = jnp.where(mask, 0.0, NEG).astype(BF)
            wq16[...] = wq_ref[...].astype(BF)
            wo16[...] = wo_ref[...].astype(BF)

        @pl.when(h == 0)
        def _():
            cpk = pltpu.make_async_copy(
                k_ref.at[b, :, pl.ds(my * HL, HL), :], k_st, dma_sem)
            cpk.start()
            cpk.wait()
            cpv = pltpu.make_async_copy(
                v_ref.at[b, :, pl.ds(my * HL, HL), :], v_st, dma_sem)
            cpv.start()
            cpv.wait()
            q16[...] = jnp.dot(x_ref[0], wq16[...],
                               preferred_element_type=jnp.float32).astype(BF)

        k16 = k_st[:, h, :].astype(BF)
        s = lax.dot_general(q16[:, pl.ds(h * DH, DH)], k16,
                            (((1,), (1,)), ((), ())),
                            preferred_element_type=jnp.float32)
        w16 = jnp.exp((s * SCALE + bias[...]).astype(BF))
        ssum = jnp.sum(w16, axis=1, keepdims=True, dtype=jnp.float32)
        recip = (1.0 / ssum).astype(BF)
        v16 = v_st[:, h, :].astype(BF)
        ctx16[:, pl.ds(h * DH, DH)] = jnp.dot(
            w16 * recip, v16, preferred_element_type=jnp.float32).astype(BF)

        @pl.when(h == HL - 1)
        def _():
            out_ref[0] = jnp.dot(ctx16[...], wo16[...],
                                 preferred_element_type=jnp.float32).astype(BF)

    return pl.pallas_call(
        body,
        grid=(B, HL),
        in_specs=[
            pl.BlockSpec((1, SQ, DM), lambda b, h: (b, 0, 0)),
            pl.BlockSpec(memory_space=pltpu.VMEM),
            pl.BlockSpec(memory_space=pltpu.HBM),
            pl.BlockSpec(memory_space=pltpu.HBM),
            pl.BlockSpec(memory_space=pltpu.VMEM),
        ],
        out_specs=pl.BlockSpec((1, SQ, DM), lambda b, h: (b, 0, 0)),
        out_shape=jax.ShapeDtypeStruct((B, SQ, DM), BF),
        scratch_shapes=[
            pltpu.VMEM((SKV, HL, DH), jnp.float32),
            pltpu.VMEM((SKV, HL, DH), jnp.float32),
            pltpu.VMEM((SQ, HL * DH), BF),
            pltpu.VMEM((SQ, HL * DH), BF),
            pltpu.VMEM((DM, HL * DH), BF),
            pltpu.VMEM((HL * DH, DM), BF),
            pltpu.VMEM((SQ, SKV), BF),
            pltpu.SemaphoreType.DMA,
        ],
        compiler_params=pltpu.CompilerParams(
            dimension_semantics=("arbitrary", "arbitrary"),
        ),
    )(x_all, Wq, K_ext, V_ext, Wo)


def _reduce_partials(partials):

    def body(p_ref, out_ref, recv_buf, send_sems, recv_sems):
        my = lax.axis_index("i")

        barrier = pltpu.get_barrier_semaphore()
        for p in range(1, N_DEV):
            peer = lax.rem(my + p, N_DEV)
            pl.semaphore_signal(barrier, inc=1, device_id=(peer,),
                                device_id_type=pl.DeviceIdType.MESH)
        pl.semaphore_wait(barrier, N_DEV - 1)

        sends = []
        for p in range(1, N_DEV):
            peer = lax.rem(my + p, N_DEV)
            rdma = pltpu.make_async_remote_copy(
                src_ref=p_ref.at[peer],
                dst_ref=recv_buf.at[my],
                send_sem=send_sems.at[peer],
                recv_sem=recv_sems.at[my],
                device_id=(peer,),
                device_id_type=pl.DeviceIdType.MESH,
            )
            rdma.start()
            sends.append(rdma)

        acc = p_ref[pl.ds(my, 1)][0].astype(jnp.float32)
        for p in range(1, N_DEV):
            j = lax.rem(my + p, N_DEV)
            recv = pltpu.make_async_remote_copy(
                src_ref=p_ref.at[j],
                dst_ref=recv_buf.at[j],
                send_sem=send_sems.at[my],
                recv_sem=recv_sems.at[j],
                device_id=(j,),
                device_id_type=pl.DeviceIdType.MESH,
            )
            recv.wait_recv()
            acc = acc + recv_buf[pl.ds(j, 1)][0].astype(jnp.float32)
        out_ref[...] = acc

        for rdma in sends:
            rdma.wait_send()

    return pl.pallas_call(
        body,
        out_shape=jax.ShapeDtypeStruct((SQ, DM), jnp.float32),
        in_specs=[pl.BlockSpec(memory_space=pltpu.VMEM)],
        out_specs=pl.BlockSpec(memory_space=pltpu.VMEM),
        scratch_shapes=[
            pltpu.VMEM((N_DEV, SQ, DM), BF),
            pltpu.SemaphoreType.DMA((N_DEV,)),
            pltpu.SemaphoreType.DMA((N_DEV,)),
        ],
        compiler_params=pltpu.CompilerParams(collective_id=1),
    )(partials)


def kernel(x, Wq, K_ext, V_ext, Wo):
    x_all = _allgather_x(x[0])
    partials = _compute_partials(x_all, Wq, K_ext, V_ext, Wo)
    out = _reduce_partials(partials)
    return out[None]
